# baseline (speedup 1.0000x reference)
import functools
import jax
import jax.numpy as jnp
from jax import lax
from jax.experimental import pallas as pl
from jax.experimental.pallas import tpu as pltpu
from jax.experimental.pallas import tpu_sc as plsc

_D = 32
_DP = 128
_V = 100000
_B = 16384
_NC = 2
_NS = 16
_NW = _NC * _NS
_BPW = _B // _NW

_mesh = plsc.VectorSubcoreMesh(core_axis_name="c", subcore_axis_name="s")


@functools.partial(
    pl.kernel,
    mesh=_mesh,
    out_type=jax.ShapeDtypeStruct((_B, 2 * _D), jnp.float32),
    scratch_types=[
        pltpu.VMEM((_BPW,), jnp.int32),
        pltpu.VMEM((_BPW,), jnp.int32),
        pltpu.VMEM((_BPW // 2, _DP), jnp.float32),
        pltpu.VMEM((_BPW, 2 * _D), jnp.float32),
        pltpu.SemaphoreType.DMA,
    ],
)
def _gather_rows(table_hbm, idx_hbm, out_hbm, idx0_v, idx1_v, rows_v, cmp_v, sem):
    wid = lax.axis_index("s") * _NC + lax.axis_index("c")
    base = wid * _BPW
    half = _BPW // 2
    pltpu.sync_copy(idx_hbm.at[0, pl.ds(base, _BPW)], idx0_v)
    pltpu.sync_copy(idx_hbm.at[1, pl.ds(base, _BPW)], idx1_v)

    def compact(row_off, dst_off):
        def body(r, carry):
            cmp_v[row_off + r, pl.ds(dst_off, 16)] = rows_v[r, pl.ds(0, 16)]
            cmp_v[row_off + r, pl.ds(dst_off + 16, 16)] = rows_v[r, pl.ds(16, 16)]
            return carry

        lax.fori_loop(0, half, body, 0, unroll=8)

    for a, idx_v in ((0, idx0_v), (1, idx1_v)):
        for c in (0, 1):
            pltpu.async_copy(
                table_hbm.at[idx_v.at[pl.ds(c * half, half)]], rows_v, sem
            ).wait()
            compact(c * half, a * _D)
    pltpu.sync_copy(cmp_v, out_hbm.at[pl.ds(base, _BPW), :])


_PAD_ROWS = 1000


def _pad_body(in_ref, out_ref):
    out_ref[:, 0:_D] = in_ref[...]


def _pad_tc(table):
    return pl.pallas_call(
        _pad_body,
        grid=(_V // _PAD_ROWS,),
        in_specs=[pl.BlockSpec((_PAD_ROWS, _D), lambda i: (i, 0))],
        out_specs=pl.BlockSpec((_PAD_ROWS, _DP), lambda i: (i, 0)),
        out_shape=jax.ShapeDtypeStruct((_V, _DP), jnp.float32),
    )(table)


def kernel(action_indices, embedding_table):
    table_p = _pad_tc(embedding_table)
    idx_t = action_indices.astype(jnp.int32).T
    return _gather_rows(table_p, idx_t)


# speedup vs baseline: 1.6160x; 1.6160x over previous
import functools
import jax
import jax.numpy as jnp
from jax import lax
from jax.experimental import pallas as pl
from jax.experimental.pallas import tpu as pltpu
from jax.experimental.pallas import tpu_sc as plsc

_D = 32
_DP = 128
_B = 16384
_NC = 2
_NS = 16
_NW = _NC * _NS
_BPW = _B // _NW

_mesh = plsc.VectorSubcoreMesh(core_axis_name="c", subcore_axis_name="s")


@functools.partial(
    pl.kernel,
    mesh=_mesh,
    out_type=jax.ShapeDtypeStruct((_B, 2 * _D), jnp.float32),
    scratch_types=[
        pltpu.VMEM((_BPW,), jnp.int32),
        pltpu.VMEM((_BPW,), jnp.int32),
        pltpu.VMEM((2, _BPW // 4, _DP), jnp.float32),
        pltpu.VMEM((_BPW, 2 * _D), jnp.float32),
        pltpu.SemaphoreType.DMA,
        pltpu.SemaphoreType.DMA,
    ],
)
def _gather_rows(table_hbm, idx_hbm, out_hbm, idx0_v, idx1_v, rows_v, cmp_v, s0, s1):
    wid = lax.axis_index("s") * _NC + lax.axis_index("c")
    base = wid * _BPW
    quarter = _BPW // 4
    pltpu.sync_copy(idx_hbm.at[0, pl.ds(base, _BPW)], idx0_v)
    pltpu.sync_copy(idx_hbm.at[1, pl.ds(base, _BPW)], idx1_v)
    sems = (s0, s1)

    def compact(slot, row_off, dst_off):
        @plsc.parallel_loop(0, quarter, unroll=8)
        def _(r):
            for h in (0, 1):
                cmp_v[row_off + r, pl.ds(dst_off + h * 16, 16)] = rows_v[
                    slot, r, pl.ds(h * 16, 16)
                ]

    chunks = [
        (idx_v, c, a * _D)
        for a, idx_v in ((0, idx0_v), (1, idx1_v))
        for c in (0, 1, 2, 3)
    ]

    def gather(i, slot):
        idx_v, c, _ = chunks[i]
        return pltpu.make_async_copy(
            table_hbm.at[idx_v.at[pl.ds(c * quarter, quarter)]], rows_v.at[slot], sems[slot]
        )

    gather(0, 0).start()
    for i in range(8):
        slot = i % 2
        gather(i, slot).wait()
        if i + 1 < 8:
            gather(i + 1, 1 - slot).start()
        _, c, dst = chunks[i]
        compact(slot, c * quarter, dst)
    pltpu.sync_copy(cmp_v, out_hbm.at[pl.ds(base, _BPW), :])


def kernel(action_indices, embedding_table):
    table_p = jnp.pad(embedding_table, ((0, 0), (0, _DP - _D)))
    idx_t = action_indices.astype(jnp.int32).T
    return _gather_rows(table_p, idx_t)
